# baseline (speedup 1.0000x reference)
import functools

import jax
import jax.numpy as jnp
from jax import lax
from jax.experimental import pallas as pl
from jax.experimental.pallas import tpu as pltpu
from jax.experimental.pallas import tpu_sc as plsc

_EPS = 0.1
_K = 5
_S = 16
_B = 64
_N = 8192
_L = 16
_CH = _N // _L
_NW = 32
_RPW = _B // _NW
_G = 8


def _insert(stack, v):
  t0, t1, t2, t3, t4 = stack
  c = v
  n0 = jnp.maximum(t0, c); c = jnp.minimum(t0, c)
  n1 = jnp.maximum(t1, c); c = jnp.minimum(t1, c)
  n2 = jnp.maximum(t2, c); c = jnp.minimum(t2, c)
  n3 = jnp.maximum(t3, c); c = jnp.minimum(t3, c)
  n4 = jnp.maximum(t4, c)
  return (n0, n1, n2, n3, n4)


def _insert2(stack, v):
  a, b = stack
  na = jnp.maximum(a, v); c = jnp.minimum(a, v)
  nb = jnp.maximum(b, c)
  return (na, nb)


def _group_update(stacks, v):
  H, A, B, C = stacks
  h0 = jnp.maximum(v[0], v[1]); l0 = jnp.minimum(v[0], v[1])
  h1 = jnp.maximum(v[2], v[3]); l1 = jnp.minimum(v[2], v[3])
  h2 = jnp.maximum(v[4], v[5]); l2 = jnp.minimum(v[4], v[5])
  h3 = jnp.maximum(v[6], v[7]); l3 = jnp.minimum(v[6], v[7])
  hh0 = jnp.maximum(h0, h1); hl0 = jnp.minimum(h0, h1)
  hh1 = jnp.maximum(h2, h3); hl1 = jnp.minimum(h2, h3)
  hhh = jnp.maximum(hh0, hh1); hhl = jnp.minimum(hh0, hh1)
  H = _insert(H, hhh)
  A = _insert2(A, hhl)
  B = _insert2(_insert2(B, hl0), hl1)
  C = _insert2(_insert2(_insert2(_insert2(C, l0), l1), l2), l3)
  return (H, A, B, C)


def _topk_body(scores_hbm, noise_hbm, out_hbm, scv,
               nb0, nb1, nb2, nb3, accv, sem0, sem1, sem2, sem3):
  wid = lax.axis_index("c") * 16 + lax.axis_index("s")
  b0 = wid * _RPW

  pltpu.sync_copy(scores_hbm.at[pl.ds(b0, _RPW)], scv)

  neg = jnp.full((_L,), -jnp.inf, jnp.float32)
  lane = lax.broadcasted_iota(jnp.int32, (_L,), 0)
  eps = jnp.float32(_EPS)
  inv_s = jnp.float32(1.0 / _S)

  def task_flat(t):
    return (t % _S) * _B + b0 + t // _S

  pltpu.async_copy(noise_hbm.at[task_flat(0)], nb0, sem0)
  pltpu.async_copy(noise_hbm.at[task_flat(1)], nb1, sem1)
  pltpu.async_copy(noise_hbm.at[task_flat(2)], nb2, sem2)
  pltpu.async_copy(noise_hbm.at[task_flat(3)], nb3, sem3)

  def top16_desc(vregs):
    cur, _ = plsc.sort_key_val(vregs[0], lane)
    for v in vregs[1:]:
      sj, _ = plsc.sort_key_val(v, lane)
      hi = jnp.maximum(cur, lax.rev(sj, (0,)))
      cur, _ = plsc.sort_key_val(hi, lane)
    return lax.rev(cur, (0,))

  def fold_stacks(stacks):
    st = stacks[0]
    for grp in stacks[1:]:
      for v in grp:
        st = _insert(st, v)
    return top16_desc(st)

  def compute_pair(r, nbx, nby, acc):
    five = tuple(neg for _ in range(_K))
    two = (neg, neg)
    init = tuple((five, two, two, two) for _ in range(4))
    half = _CH // 2

    @pl.loop(0, half, step=_G, init_carry=init, unroll=2)
    def stacks(i, carry):
      out = []
      for (nb, base), st in zip(
          ((nbx, 0), (nbx, half), (nby, 0), (nby, half)), carry):
        v = []
        for m in range(_G):
          off = (base + i + m) * _L
          v.append(scv[r, pl.ds(off, _L)] + eps * nb[pl.ds(off, _L)])
        out.append(_group_update(st, v))
      return tuple(out)

    dx = fold_stacks(stacks[0] + stacks[1][1:] + (stacks[1][0],))
    dy = fold_stacks(stacks[2] + stacks[3][1:] + (stacks[3][0],))
    return acc + jnp.where(lane < _K, (dx + dy) * inv_s, 0.0)

  for r in range(_RPW):
    @pl.loop(0, _S, step=4, init_carry=jnp.zeros((_L,), jnp.float32))
    def acc_r(s, acc_c, r=r):
      t = r * _S + s
      pltpu.make_async_copy(noise_hbm.at[0], nb0, sem0).wait()
      pltpu.make_async_copy(noise_hbm.at[0], nb1, sem1).wait()
      acc_c = compute_pair(r, nb0, nb1, acc_c)

      @pl.when(t + 4 < _RPW * _S)
      def _():
        pltpu.async_copy(noise_hbm.at[task_flat(t + 4)], nb0, sem0)
        pltpu.async_copy(noise_hbm.at[task_flat(t + 5)], nb1, sem1)

      pltpu.make_async_copy(noise_hbm.at[0], nb2, sem2).wait()
      pltpu.make_async_copy(noise_hbm.at[0], nb3, sem3).wait()
      acc_c = compute_pair(r, nb2, nb3, acc_c)

      @pl.when(t + 6 < _RPW * _S)
      def _():
        pltpu.async_copy(noise_hbm.at[task_flat(t + 6)], nb2, sem2)
        pltpu.async_copy(noise_hbm.at[task_flat(t + 7)], nb3, sem3)

      return acc_c

    accv[...] = acc_r
    pltpu.sync_copy(accv, out_hbm.at[b0 + r])


@jax.jit
def _topk_sc(scores_flat, noise_flat):
  mesh = plsc.VectorSubcoreMesh(
      core_axis_name="c", subcore_axis_name="s", num_cores=2, num_subcores=16)
  f = functools.partial(
      pl.kernel,
      out_type=jax.ShapeDtypeStruct((_B, _L), jnp.float32),
      mesh=mesh,
      compiler_params=pltpu.CompilerParams(needs_layout_passes=False),
      scratch_types=[
          pltpu.VMEM((_RPW, _N), jnp.float32),
          pltpu.VMEM((_N,), jnp.float32),
          pltpu.VMEM((_N,), jnp.float32),
          pltpu.VMEM((_N,), jnp.float32),
          pltpu.VMEM((_N,), jnp.float32),
          pltpu.VMEM((_L,), jnp.float32),
          pltpu.SemaphoreType.DMA,
          pltpu.SemaphoreType.DMA,
          pltpu.SemaphoreType.DMA,
          pltpu.SemaphoreType.DMA,
      ],
  )(_topk_body)
  return f(scores_flat, noise_flat)


def kernel(scores, noise):
  out = _topk_sc(scores, noise.reshape(_S * _B, _N))
  return out[:, :_K]


# speedup vs baseline: 1.0840x; 1.0840x over previous
import functools

import jax
import jax.numpy as jnp
from jax import lax
from jax.experimental import pallas as pl
from jax.experimental.pallas import tpu as pltpu
from jax.experimental.pallas import tpu_sc as plsc

_EPS = 0.1
_K = 5
_S = 16
_B = 64
_N = 8192
_L = 16
_CH = _N // _L
_NW = 32
_RPW = _B // _NW
_G = 8


def _insert(stack, v):
  t0, t1, t2, t3, t4 = stack
  c = v
  n0 = jnp.maximum(t0, c); c = jnp.minimum(t0, c)
  n1 = jnp.maximum(t1, c); c = jnp.minimum(t1, c)
  n2 = jnp.maximum(t2, c); c = jnp.minimum(t2, c)
  n3 = jnp.maximum(t3, c); c = jnp.minimum(t3, c)
  n4 = jnp.maximum(t4, c)
  return (n0, n1, n2, n3, n4)


def _insert2(stack, v):
  a, b = stack
  na = jnp.maximum(a, v); c = jnp.minimum(a, v)
  nb = jnp.maximum(b, c)
  return (na, nb)


def _group_update(stacks, v):
  H, A, B, C = stacks
  h0 = jnp.maximum(v[0], v[1]); l0 = jnp.minimum(v[0], v[1])
  h1 = jnp.maximum(v[2], v[3]); l1 = jnp.minimum(v[2], v[3])
  h2 = jnp.maximum(v[4], v[5]); l2 = jnp.minimum(v[4], v[5])
  h3 = jnp.maximum(v[6], v[7]); l3 = jnp.minimum(v[6], v[7])
  hh0 = jnp.maximum(h0, h1); hl0 = jnp.minimum(h0, h1)
  hh1 = jnp.maximum(h2, h3); hl1 = jnp.minimum(h2, h3)
  hhh = jnp.maximum(hh0, hh1); hhl = jnp.minimum(hh0, hh1)
  H = _insert(H, hhh)
  A = _insert2(A, hhl)
  B = _insert2(_insert2(B, hl0), hl1)
  C = _insert2(_insert2(_insert2(_insert2(C, l0), l1), l2), l3)
  return (H, A, B, C)


def _topk_body(scores_hbm, noise_hbm, out_hbm, scv,
               nb0, nb1, nb2, nb3, accv, sem0, sem1, sem2, sem3):
  wid = lax.axis_index("c") * 16 + lax.axis_index("s")
  b0 = wid * _RPW

  pltpu.sync_copy(scores_hbm.at[pl.ds(b0, _RPW)], scv)

  neg = jnp.full((_L,), -jnp.inf, jnp.float32)
  lane = lax.broadcasted_iota(jnp.int32, (_L,), 0)
  eps = jnp.float32(_EPS)
  inv_s = jnp.float32(1.0 / _S)

  def task_flat(t):
    return (t % _S) * _B + b0 + t // _S

  pltpu.async_copy(noise_hbm.at[task_flat(0)], nb0, sem0)
  pltpu.async_copy(noise_hbm.at[task_flat(1)], nb1, sem1)
  pltpu.async_copy(noise_hbm.at[task_flat(2)], nb2, sem2)
  pltpu.async_copy(noise_hbm.at[task_flat(3)], nb3, sem3)

  def top16_desc(vregs):
    cur, _ = plsc.sort_key_val(vregs[0], lane)
    for v in vregs[1:]:
      sj, _ = plsc.sort_key_val(v, lane)
      hi = jnp.maximum(cur, lax.rev(sj, (0,)))
      cur, _ = plsc.sort_key_val(hi, lane)
    return lax.rev(cur, (0,))

  def fold_stacks(stacks):
    st = stacks[0]
    for grp in stacks[1:]:
      for v in grp:
        st = _insert(st, v)
    return top16_desc(st)

  def compute_pair(r, nbx, nby, acc):
    five = tuple(neg for _ in range(_K))
    two = (neg, neg)
    init = tuple((five, two, two, two) for _ in range(2))

    @pl.loop(0, _CH, step=_G, init_carry=init, unroll=4)
    def stacks(i, carry):
      sx, sy = carry
      vx, vy = [], []
      for m in range(_G):
        off = (i + m) * _L
        sc = scv[r, pl.ds(off, _L)]
        vx.append(sc + eps * nbx[pl.ds(off, _L)])
        vy.append(sc + eps * nby[pl.ds(off, _L)])
      return (_group_update(sx, vx), _group_update(sy, vy))

    dx = fold_stacks(stacks[0])
    dy = fold_stacks(stacks[1])
    return acc + jnp.where(lane < _K, (dx + dy) * inv_s, 0.0)

  for r in range(_RPW):
    @pl.loop(0, _S, step=4, init_carry=jnp.zeros((_L,), jnp.float32))
    def acc_r(s, acc_c, r=r):
      t = r * _S + s
      pltpu.make_async_copy(noise_hbm.at[0], nb0, sem0).wait()
      pltpu.make_async_copy(noise_hbm.at[0], nb1, sem1).wait()
      acc_c = compute_pair(r, nb0, nb1, acc_c)

      @pl.when(t + 4 < _RPW * _S)
      def _():
        pltpu.async_copy(noise_hbm.at[task_flat(t + 4)], nb0, sem0)
        pltpu.async_copy(noise_hbm.at[task_flat(t + 5)], nb1, sem1)

      pltpu.make_async_copy(noise_hbm.at[0], nb2, sem2).wait()
      pltpu.make_async_copy(noise_hbm.at[0], nb3, sem3).wait()
      acc_c = compute_pair(r, nb2, nb3, acc_c)

      @pl.when(t + 6 < _RPW * _S)
      def _():
        pltpu.async_copy(noise_hbm.at[task_flat(t + 6)], nb2, sem2)
        pltpu.async_copy(noise_hbm.at[task_flat(t + 7)], nb3, sem3)

      return acc_c

    accv[...] = acc_r
    pltpu.sync_copy(accv, out_hbm.at[b0 + r])


@jax.jit
def _topk_sc(scores_flat, noise_flat):
  mesh = plsc.VectorSubcoreMesh(
      core_axis_name="c", subcore_axis_name="s", num_cores=2, num_subcores=16)
  f = functools.partial(
      pl.kernel,
      out_type=jax.ShapeDtypeStruct((_B, _L), jnp.float32),
      mesh=mesh,
      compiler_params=pltpu.CompilerParams(needs_layout_passes=False),
      scratch_types=[
          pltpu.VMEM((_RPW, _N), jnp.float32),
          pltpu.VMEM((_N,), jnp.float32),
          pltpu.VMEM((_N,), jnp.float32),
          pltpu.VMEM((_N,), jnp.float32),
          pltpu.VMEM((_N,), jnp.float32),
          pltpu.VMEM((_L,), jnp.float32),
          pltpu.SemaphoreType.DMA,
          pltpu.SemaphoreType.DMA,
          pltpu.SemaphoreType.DMA,
          pltpu.SemaphoreType.DMA,
      ],
  )(_topk_body)
  return f(scores_flat, noise_flat)


def kernel(scores, noise):
  out = _topk_sc(scores, noise.reshape(_S * _B, _N))
  return out[:, :_K]
